# baseline (speedup 1.0000x reference)
import functools

import jax
import jax.numpy as jnp
from jax.experimental import pallas as pl

N_NODE = 50000
EMB = 100
LAYERS = 2
BATCH = 1024
SEQ = 50

_BB = 128


def _attn_body(seq_h_ref, maskf_ref, slen_ref, pos_ref, w1_ref, w1b_ref,
               glu1W_ref, glu1b_ref, glu2W_ref, w2_ref, select_ref):
    seq_h = seq_h_ref[...]
    bb, L, E = seq_h.shape
    w1 = w1_ref[...]
    pos_part = pos_ref[...] @ w1[:E] + w1b_ref[...][None, :]
    hs = jnp.sum(seq_h, axis=1) / slen_ref[...]
    sh2 = seq_h.reshape(bb * L, E)
    t = sh2 @ w1[E:]
    nh = jnp.tanh(t.reshape(bb, L, E) + pos_part[None])
    g = nh.reshape(bb * L, E) @ glu1W_ref[...]
    hsg = hs @ glu2W_ref[...] + glu1b_ref[...][None, :]
    nh2 = jax.nn.sigmoid(g.reshape(bb, L, E) + hsg[:, None, :])
    beta = (nh2.reshape(bb * L, E) @ w2_ref[...]).reshape(bb, L)
    beta = beta * maskf_ref[...]
    select_ref[...] = jnp.sum(beta[:, :, None] * seq_h, axis=1)


def _attention(seq_h, maskf, session_len, pos, w1_W, w1_b, glu1_W, glu1_b,
               glu2_W, w_2):
    grid = (BATCH // _BB,)
    return pl.pallas_call(
        _attn_body,
        grid=grid,
        in_specs=[
            pl.BlockSpec((_BB, SEQ, EMB), lambda i: (i, 0, 0)),
            pl.BlockSpec((_BB, SEQ), lambda i: (i, 0)),
            pl.BlockSpec((_BB, 1), lambda i: (i, 0)),
            pl.BlockSpec((SEQ, EMB), lambda i: (0, 0)),
            pl.BlockSpec((2 * EMB, EMB), lambda i: (0, 0)),
            pl.BlockSpec((EMB,), lambda i: (0,)),
            pl.BlockSpec((EMB, EMB), lambda i: (0, 0)),
            pl.BlockSpec((EMB,), lambda i: (0,)),
            pl.BlockSpec((EMB, EMB), lambda i: (0, 0)),
            pl.BlockSpec((EMB, 1), lambda i: (0, 0)),
        ],
        out_specs=pl.BlockSpec((_BB, EMB), lambda i: (i, 0)),
        out_shape=jax.ShapeDtypeStruct((BATCH, EMB), jnp.float32),
    )(seq_h, maskf, session_len, pos, w1_W, w1_b, glu1_W, glu1_b, glu2_W, w_2)


def kernel(embedding, pos_embedding, w1_W, w1_b, w_2, glu1_W, glu1_b, glu2_W,
           adj_val, session_len, adj_idx, session_item, reversed_sess_item,
           mask):
    row = adj_idx[0]
    col = adj_idx[1]
    x = embedding
    final = embedding
    for _ in range(LAYERS):
        gathered = adj_val[:, None] * x[col]
        x = jax.ops.segment_sum(gathered, row, num_segments=N_NODE)
        final = final + x
    item_hg = final / (LAYERS + 1)

    table = jnp.concatenate([jnp.zeros((1, EMB), jnp.float32), item_hg], axis=0)
    seq_h = jnp.take(table, reversed_sess_item, axis=0)
    maskf = mask.astype(jnp.float32)

    select = _attention(seq_h, maskf, session_len, pos_embedding[:SEQ], w1_W,
                        w1_b, glu1_W, glu1_b, glu2_W, w_2)
    return (item_hg, select)
